# baseline (speedup 1.0000x reference)
import functools

import jax
import jax.numpy as jnp
from jax import lax
from jax.experimental import pallas as pl
from jax.experimental.pallas import tpu as pltpu, tpu_sc as plsc

_INFO = plsc.get_sparse_core_info()
_NC, _NS = _INFO.num_cores, _INFO.num_subcores
_NW = _NC * _NS
_CH = 8192
_HBM_FRAC_NUM, _HBM_FRAC_DEN = 29, 64


def _gather_body(n_per_w, n_hbm, n_stage, vocab, ids_hbm, table_hbm, out_hbm,
                 idx_v, rows_v, buf0, buf1, table_sh, sem, sem_h, sem_idx, s0, s1):
    sid = lax.axis_index("s")
    wid = sid * _NC + lax.axis_index("c")
    base = wid * n_per_w
    n_sp = n_per_w - n_hbm
    pltpu.async_copy(ids_hbm.at[pl.ds(base, n_per_w)], idx_v, sem_idx)

    bufs = (buf0, buf1)
    sems = (s0, s1)

    def off(j):
        return jnp.minimum((sid * n_stage + j) * _CH, vocab - _CH)

    pltpu.async_copy(table_hbm.at[pl.ds(off(0), _CH)], buf0, s0)
    pltpu.async_copy(table_hbm.at[pl.ds(off(1), _CH)], buf1, s1)
    pltpu.make_async_copy(ids_hbm.at[pl.ds(base, n_per_w)], idx_v, sem_idx).wait()
    hbm_gather = pltpu.async_copy(
        table_hbm.at[idx_v.at[pl.ds(0, n_hbm)]], rows_v.at[pl.ds(0, n_hbm)], sem_h)

    for j in range(n_stage):
        b, s = bufs[j % 2], sems[j % 2]
        pltpu.make_async_copy(table_hbm.at[pl.ds(off(j), _CH)], b, s).wait()
        pltpu.sync_copy(b, table_sh.at[pl.ds(off(j), _CH)])
        if j + 2 < n_stage:
            pltpu.async_copy(table_hbm.at[pl.ds(off(j + 2), _CH)], b, s)
    plsc.subcore_barrier()

    pltpu.async_copy(
        table_sh.at[idx_v.at[pl.ds(n_hbm, n_sp)]], rows_v.at[pl.ds(n_hbm, n_sp)],
        sem).wait()
    hbm_gather.wait()
    pltpu.sync_copy(rows_v, out_hbm.at[pl.ds(base, n_per_w)])


@functools.partial(jax.jit, static_argnames=("n_tokens", "vocab"))
def _gather_sc(token_ids, token_weights, n_tokens, vocab):
    n_per_w = n_tokens // _NW
    n_hbm = (n_per_w * _HBM_FRAC_NUM // _HBM_FRAC_DEN) & ~7
    n_stage = -(-vocab // (_NS * _CH))
    mesh = plsc.VectorSubcoreMesh(core_axis_name="c", subcore_axis_name="s")
    k = pl.kernel(
        functools.partial(_gather_body, n_per_w, n_hbm, n_stage, vocab),
        mesh=mesh,
        out_type=jax.ShapeDtypeStruct((n_tokens,), jnp.float32),
        scratch_types=[
            pltpu.VMEM((n_per_w,), jnp.int32),
            pltpu.VMEM((n_per_w,), jnp.float32),
            pltpu.VMEM((_CH,), jnp.float32),
            pltpu.VMEM((_CH,), jnp.float32),
            pltpu.VMEM_SHARED((vocab,), jnp.float32),
            pltpu.SemaphoreType.DMA,
            pltpu.SemaphoreType.DMA,
            pltpu.SemaphoreType.DMA,
            pltpu.SemaphoreType.DMA,
            pltpu.SemaphoreType.DMA,
        ],
    )
    return k(token_ids, token_weights)


def kernel(token_ids, token_weights):
    n_tokens = token_ids.shape[0]
    vocab = token_weights.shape[0]
    return _gather_sc(token_ids.astype(jnp.int32), token_weights, n_tokens, vocab)


# speedup vs baseline: 1.1166x; 1.1166x over previous
import functools

import jax
import jax.numpy as jnp
from jax import lax
from jax.experimental import pallas as pl
from jax.experimental.pallas import tpu as pltpu, tpu_sc as plsc

_INFO = plsc.get_sparse_core_info()
_NC, _NS = _INFO.num_cores, _INFO.num_subcores
_NW = _NC * _NS
_CH = 8192
_HBM_FRAC_NUM, _HBM_FRAC_DEN = 29, 64


def _gather_body(n_per_w, n_hbm, n_stage, vocab, ids_hbm, table_hbm, out_hbm,
                 idx_v, rows_v, buf0, buf1, table_sh, sem, sem_h, sem_idx, s0, s1):
    sid = lax.axis_index("s")
    wid = sid * _NC + lax.axis_index("c")
    base = wid * n_per_w
    n_sp = n_per_w - n_hbm
    pltpu.async_copy(ids_hbm.at[pl.ds(base, n_per_w)], idx_v, sem_idx)

    bufs = (buf0, buf1)
    sems = (s0, s1)

    def off(j):
        return jnp.minimum((sid * n_stage + j) * _CH, vocab - _CH)

    pltpu.async_copy(table_hbm.at[pl.ds(off(0), _CH)], buf0, s0)
    pltpu.async_copy(table_hbm.at[pl.ds(off(1), _CH)], buf1, s1)
    for j in range(n_stage):
        b, s = bufs[j % 2], sems[j % 2]
        pltpu.make_async_copy(table_hbm.at[pl.ds(off(j), _CH)], b, s).wait()
        pltpu.sync_copy(b, table_sh.at[pl.ds(off(j), _CH)])
        if j + 2 < n_stage:
            pltpu.async_copy(table_hbm.at[pl.ds(off(j + 2), _CH)], b, s)
    plsc.subcore_barrier()

    pltpu.make_async_copy(ids_hbm.at[pl.ds(base, n_per_w)], idx_v, sem_idx).wait()
    pltpu.async_copy(table_sh.at[idx_v], rows_v, sem).wait()
    pltpu.sync_copy(rows_v, out_hbm.at[pl.ds(base, n_per_w)])


@functools.partial(jax.jit, static_argnames=("n_tokens", "vocab"))
def _gather_sc(token_ids, token_weights, n_tokens, vocab):
    n_per_w = n_tokens // _NW
    n_hbm = (n_per_w * _HBM_FRAC_NUM // _HBM_FRAC_DEN) & ~7
    n_stage = -(-vocab // (_NS * _CH))
    mesh = plsc.VectorSubcoreMesh(core_axis_name="c", subcore_axis_name="s")
    k = pl.kernel(
        functools.partial(_gather_body, n_per_w, n_hbm, n_stage, vocab),
        mesh=mesh,
        out_type=jax.ShapeDtypeStruct((n_tokens,), jnp.float32),
        scratch_types=[
            pltpu.VMEM((n_per_w,), jnp.int32),
            pltpu.VMEM((n_per_w,), jnp.float32),
            pltpu.VMEM((_CH,), jnp.float32),
            pltpu.VMEM((_CH,), jnp.float32),
            pltpu.VMEM_SHARED((vocab,), jnp.float32),
            pltpu.SemaphoreType.DMA,
            pltpu.SemaphoreType.DMA,
            pltpu.SemaphoreType.DMA,
            pltpu.SemaphoreType.DMA,
            pltpu.SemaphoreType.DMA,
        ],
    )
    return k(token_ids, token_weights)


def kernel(token_ids, token_weights):
    n_tokens = token_ids.shape[0]
    vocab = token_weights.shape[0]
    return _gather_sc(token_ids.astype(jnp.int32), token_weights, n_tokens, vocab)


# speedup vs baseline: 1.1182x; 1.0014x over previous
import functools

import jax
import jax.numpy as jnp
from jax import lax
from jax.experimental import pallas as pl
from jax.experimental.pallas import tpu as pltpu, tpu_sc as plsc

_INFO = plsc.get_sparse_core_info()
_NC, _NS = _INFO.num_cores, _INFO.num_subcores
_NW = _NC * _NS
_CH = 8192
_NG = 4


def _gather_body(n_per_w, n_stage, vocab, ids_hbm, table_hbm, out_hbm,
                 idx_v, rows_v, buf0, buf1, table_sh, sem, sem_idx, s0, s1,
                 *gsems):
    sid = lax.axis_index("s")
    wid = sid * _NC + lax.axis_index("c")
    base = wid * n_per_w
    pltpu.async_copy(ids_hbm.at[pl.ds(base, n_per_w)], idx_v, sem_idx)

    bufs = (buf0, buf1)
    sems = (s0, s1)

    def off(j):
        return jnp.minimum((sid * n_stage + j) * _CH, vocab - _CH)

    pltpu.async_copy(table_hbm.at[pl.ds(off(0), _CH)], buf0, s0)
    pltpu.async_copy(table_hbm.at[pl.ds(off(1), _CH)], buf1, s1)
    for j in range(n_stage):
        b, s = bufs[j % 2], sems[j % 2]
        pltpu.make_async_copy(table_hbm.at[pl.ds(off(j), _CH)], b, s).wait()
        pltpu.sync_copy(b, table_sh.at[pl.ds(off(j), _CH)])
        if j + 2 < n_stage:
            pltpu.async_copy(table_hbm.at[pl.ds(off(j + 2), _CH)], b, s)
    plsc.subcore_barrier()

    pltpu.make_async_copy(ids_hbm.at[pl.ds(base, n_per_w)], idx_v, sem_idx).wait()
    n_sub = n_per_w // _NG
    gathers = []
    for g in range(_NG):
        gathers.append(pltpu.async_copy(
            table_sh.at[idx_v.at[pl.ds(g * n_sub, n_sub)]],
            rows_v.at[pl.ds(g * n_sub, n_sub)], gsems[g]))
    writes = []
    for g in range(_NG):
        gathers[g].wait()
        writes.append(pltpu.async_copy(
            rows_v.at[pl.ds(g * n_sub, n_sub)],
            out_hbm.at[pl.ds(base + g * n_sub, n_sub)], sem))
    for w in writes:
        w.wait()


@functools.partial(jax.jit, static_argnames=("n_tokens", "vocab"))
def _gather_sc(token_ids, token_weights, n_tokens, vocab):
    n_per_w = n_tokens // _NW
    n_stage = -(-vocab // (_NS * _CH))
    mesh = plsc.VectorSubcoreMesh(core_axis_name="c", subcore_axis_name="s")
    k = pl.kernel(
        functools.partial(_gather_body, n_per_w, n_stage, vocab),
        mesh=mesh,
        out_type=jax.ShapeDtypeStruct((n_tokens,), jnp.float32),
        scratch_types=[
            pltpu.VMEM((n_per_w,), jnp.int32),
            pltpu.VMEM((n_per_w,), jnp.float32),
            pltpu.VMEM((_CH,), jnp.float32),
            pltpu.VMEM((_CH,), jnp.float32),
            pltpu.VMEM_SHARED((vocab,), jnp.float32),
            pltpu.SemaphoreType.DMA,
            pltpu.SemaphoreType.DMA,
            pltpu.SemaphoreType.DMA,
            pltpu.SemaphoreType.DMA,
        ] + [pltpu.SemaphoreType.DMA for _ in range(_NG)],
    )
    return k(token_ids, token_weights)


def kernel(token_ids, token_weights):
    n_tokens = token_ids.shape[0]
    vocab = token_weights.shape[0]
    return _gather_sc(token_ids.astype(jnp.int32), token_weights, n_tokens, vocab)


# speedup vs baseline: 1.1276x; 1.0084x over previous
import functools

import jax
import jax.numpy as jnp
from jax import lax
from jax.experimental import pallas as pl
from jax.experimental.pallas import tpu as pltpu, tpu_sc as plsc

_INFO = plsc.get_sparse_core_info()
_NC, _NS = _INFO.num_cores, _INFO.num_subcores
_NW = _NC * _NS
_CH = 8192
_NG = 4


def _gather_body(n_per_w, n_stage, vocab, ids_hbm, table_hbm, out_hbm,
                 idx_v, rows_v, buf0, buf1, table_sh, sem, sem_idx, s0, s1,
                 *gsems):
    sid = lax.axis_index("s")
    wid = sid * _NC + lax.axis_index("c")
    base = wid * n_per_w
    pltpu.async_copy(ids_hbm.at[pl.ds(base, n_per_w)], idx_v, sem_idx)

    bufs = (buf0, buf1)
    sems = (s0, s1)

    def off(j):
        return jnp.minimum((sid * n_stage + j) * _CH, vocab - _CH)

    with jax.named_scope("stage_table"):
        pltpu.async_copy(table_hbm.at[pl.ds(off(0), _CH)], buf0, s0)
        pltpu.async_copy(table_hbm.at[pl.ds(off(1), _CH)], buf1, s1)
        for j in range(n_stage):
            b, s = bufs[j % 2], sems[j % 2]
            pltpu.make_async_copy(table_hbm.at[pl.ds(off(j), _CH)], b, s).wait()
            pltpu.sync_copy(b, table_sh.at[pl.ds(off(j), _CH)])
            if j + 2 < n_stage:
                pltpu.async_copy(table_hbm.at[pl.ds(off(j + 2), _CH)], b, s)
    with jax.named_scope("barrier"):
        plsc.subcore_barrier()

    with jax.named_scope("idx_wait"):
        pltpu.make_async_copy(ids_hbm.at[pl.ds(base, n_per_w)], idx_v, sem_idx).wait()
    n_sub = n_per_w // _NG
    with jax.named_scope("gather"):
        gathers = []
        for g in range(_NG):
            gathers.append(pltpu.async_copy(
                table_sh.at[idx_v.at[pl.ds(g * n_sub, n_sub)]],
                rows_v.at[pl.ds(g * n_sub, n_sub)], gsems[g]))
        writes = []
        for g in range(_NG):
            gathers[g].wait()
            writes.append(pltpu.async_copy(
                rows_v.at[pl.ds(g * n_sub, n_sub)],
                out_hbm.at[pl.ds(base + g * n_sub, n_sub)], sem))
    with jax.named_scope("drain_writes"):
        for w in writes:
            w.wait()


@functools.partial(jax.jit, static_argnames=("n_tokens", "vocab"))
def _gather_sc(token_ids, token_weights, n_tokens, vocab):
    n_per_w = n_tokens // _NW
    n_stage = -(-vocab // (_NS * _CH))
    mesh = plsc.VectorSubcoreMesh(core_axis_name="c", subcore_axis_name="s")
    k = pl.kernel(
        functools.partial(_gather_body, n_per_w, n_stage, vocab),
        mesh=mesh,
        out_type=jax.ShapeDtypeStruct((n_tokens,), jnp.float32),
        scratch_types=[
            pltpu.VMEM((n_per_w,), jnp.int32),
            pltpu.VMEM((n_per_w,), jnp.float32),
            pltpu.VMEM((_CH,), jnp.float32),
            pltpu.VMEM((_CH,), jnp.float32),
            pltpu.VMEM_SHARED((vocab,), jnp.float32),
            pltpu.SemaphoreType.DMA,
            pltpu.SemaphoreType.DMA,
            pltpu.SemaphoreType.DMA,
            pltpu.SemaphoreType.DMA,
        ] + [pltpu.SemaphoreType.DMA for _ in range(_NG)],
    )
    return k(token_ids, token_weights)


def kernel(token_ids, token_weights):
    n_tokens = token_ids.shape[0]
    vocab = token_weights.shape[0]
    return _gather_sc(token_ids.astype(jnp.int32), token_weights, n_tokens, vocab)
